# baseline (speedup 1.0000x reference)
import functools

import jax
import jax.numpy as jnp
from jax import lax
from jax.experimental import pallas as pl
from jax.experimental.pallas import tpu as pltpu
from jax.experimental.pallas import tpu_sc as plsc

_IDX_CHUNK = 128
_NSEM = 8


def _gather_rows(w_hbm, idx_vmem, rows_vmem, sems, n):
    lanes = 16
    nsem = len(sems)
    assert n % lanes == 0 and lanes % nsem == 0

    def fire(g, _):
        vec = idx_vmem[pl.ds(g * lanes, lanes)]
        for j in range(lanes):
            pltpu.async_copy(w_hbm.at[pl.ds(vec[j], 1)],
                             rows_vmem.at[pl.ds(g * lanes + j, 1)],
                             sems[j % nsem])
        return 0

    lax.fori_loop(0, n // lanes, fire, 0)
    per_sem = n // nsem
    for k in range(nsem):
        pltpu.make_async_copy(
            w_hbm.at[pl.ds(0, per_sem)],
            rows_vmem.at[pl.ds(k * per_sem, per_sem)],
            sems[k]).wait()


def _sc_gather(W, bias, tids, sids):
    B = tids.shape[0]
    S = sids.shape[0]
    D = W.shape[1]
    info = plsc.get_sparse_core_info()
    nc, ns = info.num_cores, info.num_subcores
    nw = nc * ns
    bt = B // nw
    bs = S // nw

    mesh = plsc.VectorSubcoreMesh(core_axis_name="c", subcore_axis_name="s")

    @functools.partial(
        pl.kernel,
        mesh=mesh,
        out_type=(
            jax.ShapeDtypeStruct((B, D), jnp.float32),
            jax.ShapeDtypeStruct((B,), jnp.float32),
            jax.ShapeDtypeStruct((S, D), jnp.float32),
            jax.ShapeDtypeStruct((S,), jnp.float32),
        ),
        scratch_types=[
            pltpu.VMEM((bt,), jnp.int32),
            pltpu.VMEM((bt, D), jnp.float32),
            pltpu.VMEM((bt,), jnp.float32),
            pltpu.VMEM((bs,), jnp.int32),
            pltpu.VMEM((bs, D), jnp.float32),
            pltpu.VMEM((bs,), jnp.float32),
            pltpu.SemaphoreType.DMA,
        ] + [pltpu.SemaphoreType.DMA] * _NSEM,
    )
    def sc_kernel(w_hbm, b_hbm, tid_hbm, sid_hbm,
                  tw_hbm, tb_hbm, sw_hbm, sb_hbm,
                  tidx_v, trows, tbv, sidx_v, srows, sbv,
                  bsem, *sems):
        wid = lax.axis_index("s") * nc + lax.axis_index("c")

        tbase = wid * bt
        pltpu.sync_copy(tid_hbm.at[pl.ds(tbase, bt)], tidx_v)
        bias_copies = []
        for j in range(bt // _IDX_CHUNK):
            sl = pl.ds(j * _IDX_CHUNK, _IDX_CHUNK)
            bias_copies.append(
                pltpu.async_copy(b_hbm.at[tidx_v.at[sl]], tbv.at[sl], bsem))
        _gather_rows(w_hbm, tidx_v, trows, sems, bt)
        for c in bias_copies:
            c.wait()
        pltpu.sync_copy(trows, tw_hbm.at[pl.ds(tbase, bt)])
        pltpu.sync_copy(tbv, tb_hbm.at[pl.ds(tbase, bt)])

        sbase = wid * bs
        pltpu.sync_copy(sid_hbm.at[pl.ds(sbase, bs)], sidx_v)
        bc = pltpu.async_copy(b_hbm.at[sidx_v], sbv, bsem)
        _gather_rows(w_hbm, sidx_v, srows, sems, bs)
        bc.wait()
        pltpu.sync_copy(srows, sw_hbm.at[pl.ds(sbase, bs)])
        pltpu.sync_copy(sbv, sb_hbm.at[pl.ds(sbase, bs)])

    return sc_kernel(W, bias, tids, sids)


def _tc_logits(x, tw, tb, sw_aug, bm=1024):
    B, D = x.shape
    N, K = sw_aug.shape

    def body(x_ref, tw_ref, tb_ref, swa_ref, out_ref):
        xb = x_ref[...]
        tl = jnp.sum(xb * tw_ref[...], axis=1, keepdims=True)
        tl = tl + tb_ref[...].reshape(bm, 1)
        ones = jnp.ones((bm, 1), jnp.float32)
        x_aug = jnp.concatenate([xb, tl, ones], axis=1)
        out_ref[...] = lax.dot_general(
            x_aug, swa_ref[...], (((1,), (1,)), ((), ())),
            preferred_element_type=jnp.float32)

    return pl.pallas_call(
        body,
        grid=(B // bm,),
        in_specs=[
            pl.BlockSpec((bm, D), lambda i: (i, 0)),
            pl.BlockSpec((bm, D), lambda i: (i, 0)),
            pl.BlockSpec((bm,), lambda i: (i,)),
            pl.BlockSpec((N, K), lambda i: (0, 0)),
        ],
        out_specs=pl.BlockSpec((bm, N), lambda i: (i, 0)),
        out_shape=jax.ShapeDtypeStruct((B, N), jnp.float32),
    )(x, tw, tb, sw_aug)


def kernel(inputs, W, bias, target, sampled):
    tids = target.reshape(-1)
    tw, tb, sw, sb = _sc_gather(W, bias, tids, sampled)
    S, D = sw.shape
    rows = jnp.concatenate(
        [sw, jnp.zeros((S, 1), jnp.float32), sb[:, None]], axis=1)
    row0 = jnp.zeros((1, D + 2), jnp.float32).at[0, D].set(1.0)
    sw_aug = jnp.concatenate([row0, rows], axis=0)
    return _tc_logits(inputs, tw, tb, sw_aug)
